# baseline (speedup 1.0000x reference)
import functools

import jax
import jax.numpy as jnp
from jax import lax
from jax.experimental import pallas as pl
from jax.experimental.pallas import tpu as pltpu
from jax.experimental.pallas import tpu_sc as plsc

_NL, _NO, _NA = 50000, 5000, 20000
_E1, _E2 = 256000, 128000
_DL, _DC = 128, 256

_NC, _NS = 2, 16
_B = 128
_OPAD = 5120
_AQ = 5000
_AQPAD = 5120


def _mesh():
    return plsc.VectorSubcoreMesh(core_axis_name="c", subcore_axis_name="s")


_SC_PARAMS = pltpu.CompilerParams(use_tc_tiling_on_sc=False)


def _sc_counts(org_idx, abn_idx, zeros_flat):
    ep1 = _E1 // (_NC * _NS)
    ep2 = _E2 // (_NC * _NS)
    nb1 = ep1 // _B
    nb2 = ep2 // _B
    n2 = _AQPAD * 4

    @functools.partial(
        pl.kernel,
        out_type=[jax.ShapeDtypeStruct((32, _OPAD), jnp.float32),
                  jax.ShapeDtypeStruct((32, n2), jnp.float32)],
        mesh=_mesh(),
        scratch_types=[pltpu.VMEM((_B,), jnp.int32),
                       pltpu.VMEM((_OPAD,), jnp.float32),
                       pltpu.VMEM((n2,), jnp.float32)],
        compiler_params=_SC_PARAMS,
    )
    def k(oidx_hbm, aidx_hbm, zs_hbm, out_c1, out_c2, idx_v, acc1, acc2):
        c = lax.axis_index("c")
        s = lax.axis_index("s")
        wid = c * _NS + s
        lanes = lax.iota(jnp.int32, 16)
        pltpu.sync_copy(zs_hbm.at[pl.ds(0, _OPAD)], acc1)
        pltpu.sync_copy(zs_hbm.at[pl.ds(0, n2)], acc2)

        @pl.loop(0, nb1)
        def _(i):
            pltpu.sync_copy(oidx_hbm.at[pl.ds(wid * ep1 + i * _B, _B)],
                            idx_v)

            @pl.loop(0, _B // 16)
            def _(e16):
                vec = idx_v[pl.ds(e16 * 16, 16)]
                for u in range(16):
                    o = vec[u]
                    hot = jnp.where(lanes == (o & 15), 1.0, 0.0)
                    r = pl.multiple_of((o >> 4) * 16, 16)
                    acc1[pl.ds(r, 16)] = acc1[pl.ds(r, 16)] + hot

        @pl.loop(0, nb2)
        def _(i):
            pltpu.sync_copy(aidx_hbm.at[pl.ds(wid * ep2 + i * _B, _B)],
                            idx_v)

            @pl.loop(0, _B // 16)
            def _(e16):
                vec = idx_v[pl.ds(e16 * 16, 16)]
                for u in range(16):
                    a = vec[u]
                    hot = jnp.where(lanes == (a & 15), 1.0, 0.0)
                    r = pl.multiple_of((a >> 4) * 16, 16)
                    acc2[pl.ds(r, 16)] = acc2[pl.ds(r, 16)] + hot

        pltpu.sync_copy(acc1, out_c1.at[wid])
        pltpu.sync_copy(acc2, out_c2.at[wid])

    return k(org_idx, abn_idx, zeros_flat)


def _sc_org_segsum(lab_enh, rel_proj, lab_idx, org_idx, zeros_slc):
    epg = _E1 // 4
    nblk = epg // _B

    @functools.partial(
        pl.kernel,
        out_type=jax.ShapeDtypeStruct((4, 8, _OPAD, 16), jnp.float32),
        mesh=_mesh(),
        scratch_types=[pltpu.VMEM((_B,), jnp.int32),
                       pltpu.VMEM((_B,), jnp.int32),
                       pltpu.VMEM((_B,), jnp.int32),
                       pltpu.VMEM((_B,), jnp.int32),
                       pltpu.VMEM((_B, 16), jnp.float32),
                       pltpu.VMEM((_B, 16), jnp.float32),
                       pltpu.VMEM((_OPAD, 16), jnp.float32),
                       pltpu.SemaphoreType.DMA],
        compiler_params=_SC_PARAMS,
    )
    def k(tab_hbm, rel_hbm, lidx_hbm, oidx_hbm, zs_hbm, out_sum,
          lidx_v, oidx_v, gi_v, ri_v, lab_v, rel_v, acc, sem):
        c = lax.axis_index("c")
        s = lax.axis_index("s")
        wid = c * _NS + s
        f = wid % 8
        g = wid // 8
        lanes8 = lax.iota(jnp.int32, 16) * 8
        pltpu.sync_copy(zs_hbm, acc)
        base = g * epg

        @pl.loop(0, nblk)
        def _(i):
            off = base + i * _B
            pltpu.sync_copy(lidx_hbm.at[pl.ds(off, _B)], lidx_v)
            pltpu.sync_copy(oidx_hbm.at[pl.ds(off, _B)], oidx_v)

            @pl.loop(0, _B // 16)
            def _(c16):
                lvec = lidx_v[pl.ds(c16 * 16, 16)]
                gi_v[pl.ds(c16 * 16, 16)] = lvec * 8 + f
                ri_v[pl.ds(c16 * 16, 16)] = ((off + c16 * 16) * 8 + f
                                             + lanes8)

            cop_a = pltpu.async_copy(tab_hbm.at[gi_v], lab_v, sem)
            cop_b = pltpu.async_copy(rel_hbm.at[ri_v], rel_v, sem)
            cop_a.wait()
            cop_b.wait()

            @pl.loop(0, _B // 16)
            def _(e16):
                vec = oidx_v[pl.ds(e16 * 16, 16)]
                for u in range(16):
                    e = e16 * 16 + u
                    o = vec[u]
                    acc[o, :] = acc[o, :] + (lab_v[e, :] + rel_v[e, :])

        pltpu.sync_copy(acc, out_sum.at[g].at[f])

    return k(lab_enh, rel_proj, lab_idx, org_idx, zeros_slc)


def _sc_abn_segsum(org_tab, org_idx, abn_idx, zeros_slc):
    nblk = _E2 // _B

    @functools.partial(
        pl.kernel,
        out_type=jax.ShapeDtypeStruct((4, 8, _AQPAD, 16), jnp.float32),
        mesh=_mesh(),
        scratch_types=[pltpu.VMEM((_B,), jnp.int32),
                       pltpu.VMEM((_B,), jnp.int32),
                       pltpu.VMEM((_B,), jnp.int32),
                       pltpu.VMEM((_B, 16), jnp.float32),
                       pltpu.VMEM((_AQPAD, 16), jnp.float32),
                       pltpu.SemaphoreType.DMA],
        compiler_params=_SC_PARAMS,
    )
    def k(tab_hbm, oidx_hbm, aidx_hbm, zs_hbm, out_sum,
          oidx_v, aidx_v, gi_v, row_v, acc, sem):
        c = lax.axis_index("c")
        s = lax.axis_index("s")
        wid = c * _NS + s
        f = wid % 8
        q = wid // 8
        abase = q * _AQ
        pltpu.sync_copy(zs_hbm, acc)

        @pl.loop(0, nblk)
        def _(i):
            off = i * _B
            pltpu.sync_copy(oidx_hbm.at[pl.ds(off, _B)], oidx_v)
            pltpu.sync_copy(aidx_hbm.at[pl.ds(off, _B)], aidx_v)

            @pl.loop(0, _B // 16)
            def _(c16):
                ovec = oidx_v[pl.ds(c16 * 16, 16)]
                gi_v[pl.ds(c16 * 16, 16)] = ovec * 8 + f

            pltpu.async_copy(tab_hbm.at[gi_v], row_v, sem).wait()

            @pl.loop(0, _B // 16)
            def _(e16):
                vec = aidx_v[pl.ds(e16 * 16, 16)]
                for u in range(16):
                    e = e16 * 16 + u
                    a = vec[u] - abase
                    owned = (a >= 0) & (a < _AQ)
                    t = jnp.where(owned, a, _AQ)
                    acc[t, :] = acc[t, :] + row_v[e, :]

        pltpu.sync_copy(acc, out_sum.at[q].at[f])

    return k(org_tab, org_idx, abn_idx, zeros_slc)


def _tc_lab_enh(lab_feats, lab_concept, w1t, w2t, b):
    blk = 1000
    grid = _NL // blk

    def body(x1, x2, w1, w2, bb, o):
        acc = jnp.dot(x1[...], w1[...], preferred_element_type=jnp.float32)
        acc = acc + jnp.dot(x2[...], w2[...],
                            preferred_element_type=jnp.float32)
        o[...] = jnp.maximum(acc + bb[...], 0.0)

    return pl.pallas_call(
        body,
        grid=(grid,),
        in_specs=[pl.BlockSpec((blk, _DL), lambda i: (i, 0)),
                  pl.BlockSpec((blk, _DC), lambda i: (i, 0)),
                  pl.BlockSpec((_DL, _DL), lambda i: (0, 0)),
                  pl.BlockSpec((_DC, _DL), lambda i: (0, 0)),
                  pl.BlockSpec((1, _DL), lambda i: (0, 0))],
        out_specs=pl.BlockSpec((blk, _DL), lambda i: (i, 0)),
        out_shape=jax.ShapeDtypeStruct((_NL, _DL), jnp.float32),
    )(lab_feats, lab_concept, w1t, w2t, b)


def _tc_rel_proj(rel_emb, dt):
    blk = 2000
    grid = _E1 // blk

    def body(x, d, o):
        o[...] = jnp.dot(x[...], d[...], preferred_element_type=jnp.float32)

    return pl.pallas_call(
        body,
        grid=(grid,),
        in_specs=[pl.BlockSpec((blk, _DC), lambda i: (i, 0)),
                  pl.BlockSpec((_DC, _DL), lambda i: (0, 0))],
        out_specs=pl.BlockSpec((blk, _DL), lambda i: (i, 0)),
        out_shape=jax.ShapeDtypeStruct((_E1, _DL), jnp.float32),
    )(rel_emb, dt)


def _tc_org(parts, cnts, cnts2, wot, wo2t, b1, b2):
    def body(pp, cp, cp2, wo, w2, bb1, bb2, o, o2):
        org_sum = (pp[0, :_NO, :] + pp[1, :_NO, :]
                   + pp[2, :_NO, :] + pp[3, :_NO, :])
        cnt = jnp.sum(cp[...], axis=0)[:_NO, None]
        org_agg = org_sum / jnp.maximum(cnt, 1.0)
        st = jnp.maximum(
            jnp.dot(org_agg, wo[...], preferred_element_type=jnp.float32)
            + bb1[...], 0.0)
        o[...] = jnp.dot(st, w2[...],
                         preferred_element_type=jnp.float32) + bb2[...]
        o2[...] = jnp.sum(cp2[...], axis=0)[:, None]

    return pl.pallas_call(
        body,
        out_shape=[jax.ShapeDtypeStruct((_NO, _DL), jnp.float32),
                   jax.ShapeDtypeStruct((_AQPAD * 4, 1), jnp.float32)],
    )(parts, cnts, cnts2, wot, wo2t, b1, b2)


def _tc_abn(abn_part, cnt_part, abn_feats, abn_concept, a1t, a2t, b):
    blk = 1000
    grid = _NA // blk
    per_q = _AQ // blk

    def body(ap, cp, ff, cc, w1, w2, bb, o):
        msg = ap[0] / jnp.maximum(cp[...], 1.0)
        x = ff[...] + msg
        o[...] = (jnp.dot(x, w1[...], preferred_element_type=jnp.float32)
                  + jnp.dot(cc[...], w2[...],
                            preferred_element_type=jnp.float32)
                  + bb[...])

    return pl.pallas_call(
        body,
        grid=(grid,),
        in_specs=[
            pl.BlockSpec((1, blk, _DL), lambda i: (i // per_q, i % per_q, 0)),
            pl.BlockSpec((blk, 1), lambda i: (i, 0)),
            pl.BlockSpec((blk, _DL), lambda i: (i, 0)),
            pl.BlockSpec((blk, _DC), lambda i: (i, 0)),
            pl.BlockSpec((_DL, _DL), lambda i: (0, 0)),
            pl.BlockSpec((_DC, _DL), lambda i: (0, 0)),
            pl.BlockSpec((1, _DL), lambda i: (0, 0)),
        ],
        out_specs=pl.BlockSpec((blk, _DL), lambda i: (i, 0)),
        out_shape=jax.ShapeDtypeStruct((_NA, _DL), jnp.float32),
    )(abn_part, cnt_part, abn_feats, abn_concept, a1t, a2t, b)


def kernel(lab_feats, abn_feats, lab_concept, abn_concept, lab_org_rel_emb,
           lab_org_lab_idx, lab_org_org_idx, o2a_abn_idx, o2a_org_idx,
           W_lab_w, W_lab_b, W_abn_w, W_abn_b, W_org_w, W_org_b,
           D_w, W_o2a_w, W_o2a_b):
    f32 = jnp.float32
    w1t = W_lab_w[:, :_DL].T
    w2t = W_lab_w[:, _DL:].T
    a1t = W_abn_w[:, :_DL].T
    a2t = W_abn_w[:, _DL:].T
    dt = D_w.T
    wot = W_org_w.T
    wo2t = W_o2a_w.T
    b_lab = W_lab_b.reshape(1, _DL)
    b_org = W_org_b.reshape(1, _DL)
    b_o2a = W_o2a_b.reshape(1, _DL)
    b_abn = W_abn_b.reshape(1, _DL)

    z_slc = jnp.zeros((_OPAD, 16), f32)
    z_flat = jnp.zeros((_AQPAD * 4,), f32)

    c1, c2 = _sc_counts(lab_org_org_idx, o2a_abn_idx, z_flat)
    lab_enh = _tc_lab_enh(lab_feats, lab_concept, w1t, w2t, b_lab)
    rel_proj = _tc_rel_proj(lab_org_rel_emb, dt)
    parts = _sc_org_segsum(lab_enh.reshape(_NL * 8, 16),
                           rel_proj.reshape(_E1 * 8, 16),
                           lab_org_lab_idx, lab_org_org_idx, z_slc)
    parts = parts.transpose(0, 2, 1, 3).reshape(4, _OPAD, _DL)
    org_tab, cnt2 = _tc_org(parts, c1, c2, wot, wo2t, b_org, b_o2a)
    abn_part = _sc_abn_segsum(org_tab.reshape(_NO * 8, 16),
                              o2a_org_idx, o2a_abn_idx, z_slc)
    abn_part = abn_part.transpose(0, 2, 1, 3).reshape(4, _AQPAD, _DL)
    return _tc_abn(abn_part, cnt2, abn_feats, abn_concept, a1t, a2t, b_abn)


# speedup vs baseline: 2.5441x; 2.5441x over previous
import functools

import jax
import jax.numpy as jnp
from jax import lax
from jax.experimental import pallas as pl
from jax.experimental.pallas import tpu as pltpu
from jax.experimental.pallas import tpu_sc as plsc

_NL, _NO, _NA = 50000, 5000, 20000
_E1, _E2 = 256000, 128000
_DL, _DC = 128, 256

_NC, _NS = 2, 16
_SUB = 128
_NSUB = 10
_CH = _SUB * _NSUB
_OPAD = 5120
_AQ = 5000
_AQPAD = 5120
_ACCW = _OPAD * 16


def _mesh():
    return plsc.VectorSubcoreMesh(core_axis_name="c", subcore_axis_name="s")


_SC_PARAMS = pltpu.CompilerParams(use_tc_tiling_on_sc=False,
                                  needs_layout_passes=False)


def _sc_counts(org_idx, abn_idx, zeros_flat):
    ep1 = _E1 // (_NC * _NS)
    ep2 = _E2 // (_NC * _NS)
    ch = 2000
    nb1 = ep1 // ch
    nb2 = ep2 // ch
    n2 = _AQPAD * 4

    @functools.partial(
        pl.kernel,
        out_type=[jax.ShapeDtypeStruct((32, _OPAD), jnp.float32),
                  jax.ShapeDtypeStruct((32, n2), jnp.float32)],
        mesh=_mesh(),
        scratch_types=[pltpu.VMEM((ch,), jnp.int32),
                       pltpu.VMEM((_OPAD,), jnp.float32),
                       pltpu.VMEM((n2,), jnp.float32)],
        compiler_params=_SC_PARAMS,
    )
    def k(oidx_hbm, aidx_hbm, zs_hbm, out_c1, out_c2, idx_v, acc1, acc2):
        c = lax.axis_index("c")
        s = lax.axis_index("s")
        wid = c * _NS + s
        lanes = lax.iota(jnp.int32, 16)
        mask0 = lanes == 0
        ones = jnp.ones((16,), jnp.float32)
        pltpu.sync_copy(zs_hbm.at[pl.ds(0, _OPAD)], acc1)
        pltpu.sync_copy(zs_hbm.at[pl.ds(0, n2)], acc2)

        def count_into(acc, idx_hbm, ep, nb):
            @pl.loop(0, nb)
            def _(i):
                pltpu.sync_copy(idx_hbm.at[pl.ds(wid * ep + i * ch, ch)],
                                idx_v)

                @pl.loop(0, ch // 16)
                def _(e16):
                    vec = idx_v[pl.ds(e16 * 16, 16)]
                    for u in range(16):
                        offs = vec[u] + jnp.zeros((16,), jnp.int32)
                        plsc.addupdate_scatter(acc, [offs], ones, mask=mask0)

        count_into(acc1, oidx_hbm, ep1, nb1)
        count_into(acc2, aidx_hbm, ep2, nb2)
        pltpu.sync_copy(acc1, out_c1.at[wid])
        pltpu.sync_copy(acc2, out_c2.at[wid])

    return k(org_idx, abn_idx, zeros_flat)


def _sc_org_segsum(lab_enh, rel_proj, lab_idx, org_idx, zeros_flat):
    epg = _E1 // 4
    nch = epg // _CH

    @functools.partial(
        pl.kernel,
        out_type=jax.ShapeDtypeStruct((4, 8, _ACCW), jnp.float32),
        mesh=_mesh(),
        scratch_types=[pltpu.VMEM((2, _CH), jnp.int32),
                       pltpu.VMEM((2, _CH), jnp.int32),
                       pltpu.VMEM((2, _SUB), jnp.int32),
                       pltpu.VMEM((2, _SUB), jnp.int32),
                       pltpu.VMEM((2, _SUB, 16), jnp.float32),
                       pltpu.VMEM((2, _SUB, 16), jnp.float32),
                       pltpu.VMEM((_ACCW,), jnp.float32),
                       pltpu.SemaphoreType.DMA,
                       pltpu.SemaphoreType.DMA,
                       pltpu.SemaphoreType.DMA,
                       pltpu.SemaphoreType.DMA],
        compiler_params=_SC_PARAMS,
    )
    def k(tab_hbm, rel_hbm, lidx_hbm, oidx_hbm, zs_hbm, out_sum,
          lidx_v, oidx_v, gi_v, ri_v, lab_v, rel_v, acc,
          si0, si1, sg0, sg1):
        c = lax.axis_index("c")
        s = lax.axis_index("s")
        wid = c * _NS + s
        f = wid % 8
        g = wid // 8
        lanes = lax.iota(jnp.int32, 16)
        lanes8 = lanes * 8
        pltpu.sync_copy(zs_hbm, acc)
        base = g * epg
        si = (si0, si1)
        sg = (sg0, sg1)

        def issue_idx(chv, b):
            off = base + chv * _CH
            pltpu.async_copy(lidx_hbm.at[pl.ds(off, _CH)], lidx_v.at[b],
                             si[b])
            pltpu.async_copy(oidx_hbm.at[pl.ds(off, _CH)], oidx_v.at[b],
                             si[b])

        def wait_idx(b):
            pltpu.make_async_copy(lidx_hbm.at[pl.ds(0, _CH)],
                                  lidx_v.at[b], si[b]).wait()
            pltpu.make_async_copy(oidx_hbm.at[pl.ds(0, _CH)],
                                  oidx_v.at[b], si[b]).wait()

        def build_issue(chv, b, j, gb):
            off = base + chv * _CH + j * _SUB

            @pl.loop(0, _SUB // 16)
            def _(c16):
                lvec = lidx_v[b, pl.ds(j * _SUB + c16 * 16, 16)]
                gi_v[gb, pl.ds(c16 * 16, 16)] = lvec * 8 + f
                ri_v[gb, pl.ds(c16 * 16, 16)] = ((off + c16 * 16) * 8 + f
                                                 + lanes8)

            pltpu.async_copy(tab_hbm.at[gi_v.at[gb]], lab_v.at[gb], sg[gb])
            pltpu.async_copy(rel_hbm.at[ri_v.at[gb]], rel_v.at[gb], sg[gb])

        def wait_process(b, j, gb):
            pltpu.make_async_copy(tab_hbm.at[pl.ds(0, _SUB)],
                                  lab_v.at[gb], sg[gb]).wait()
            pltpu.make_async_copy(rel_hbm.at[pl.ds(0, _SUB)],
                                  rel_v.at[gb], sg[gb]).wait()

            @pl.loop(0, _SUB // 16)
            def _(e16):
                vec = oidx_v[b, pl.ds(j * _SUB + e16 * 16, 16)]
                for u in range(16):
                    e = e16 * 16 + u
                    vals = lab_v[gb, e, :] + rel_v[gb, e, :]
                    offs = vec[u] * 16 + lanes
                    plsc.addupdate_scatter(acc, [offs], vals)

        issue_idx(0, 0)
        issue_idx(1, 1)

        @pl.loop(0, nch, step=2)
        def _(chv):
            for b in (0, 1):
                cur = chv + b
                wait_idx(b)
                build_issue(cur, b, 0, 0)
                for j in range(1, _NSUB):
                    build_issue(cur, b, j, j % 2)
                    wait_process(b, j - 1, (j - 1) % 2)
                wait_process(b, _NSUB - 1, (_NSUB - 1) % 2)

                @pl.when(cur + 2 < nch)
                def _():
                    issue_idx(cur + 2, b)

        pltpu.sync_copy(acc, out_sum.at[g].at[f])

    return k(lab_enh, rel_proj, lab_idx, org_idx, zeros_flat)


def _sc_abn_segsum(org_tab, org_idx, abn_idx, zeros_flat):
    nch = _E2 // _CH

    @functools.partial(
        pl.kernel,
        out_type=jax.ShapeDtypeStruct((4, 8, _ACCW), jnp.float32),
        mesh=_mesh(),
        scratch_types=[pltpu.VMEM((2, _CH), jnp.int32),
                       pltpu.VMEM((2, _CH), jnp.int32),
                       pltpu.VMEM((2, _SUB), jnp.int32),
                       pltpu.VMEM((2, _SUB, 16), jnp.float32),
                       pltpu.VMEM((_ACCW,), jnp.float32),
                       pltpu.SemaphoreType.DMA,
                       pltpu.SemaphoreType.DMA,
                       pltpu.SemaphoreType.DMA,
                       pltpu.SemaphoreType.DMA],
        compiler_params=_SC_PARAMS,
    )
    def k(tab_hbm, oidx_hbm, aidx_hbm, zs_hbm, out_sum,
          oidx_v, aidx_v, gi_v, row_v, acc, si0, si1, sg0, sg1):
        c = lax.axis_index("c")
        s = lax.axis_index("s")
        wid = c * _NS + s
        f = wid % 8
        q = wid // 8
        abase = q * _AQ
        lanes = lax.iota(jnp.int32, 16)
        pltpu.sync_copy(zs_hbm, acc)
        si = (si0, si1)
        sg = (sg0, sg1)

        def issue_idx(chv, b):
            off = chv * _CH
            pltpu.async_copy(oidx_hbm.at[pl.ds(off, _CH)], oidx_v.at[b],
                             si[b])
            pltpu.async_copy(aidx_hbm.at[pl.ds(off, _CH)], aidx_v.at[b],
                             si[b])

        def wait_idx(b):
            pltpu.make_async_copy(oidx_hbm.at[pl.ds(0, _CH)],
                                  oidx_v.at[b], si[b]).wait()
            pltpu.make_async_copy(aidx_hbm.at[pl.ds(0, _CH)],
                                  aidx_v.at[b], si[b]).wait()

        def build_issue(chv, b, j, gb):
            @pl.loop(0, _SUB // 16)
            def _(c16):
                ovec = oidx_v[b, pl.ds(j * _SUB + c16 * 16, 16)]
                gi_v[gb, pl.ds(c16 * 16, 16)] = ovec * 8 + f

            pltpu.async_copy(tab_hbm.at[gi_v.at[gb]], row_v.at[gb], sg[gb])

        def wait_process(b, j, gb):
            pltpu.make_async_copy(tab_hbm.at[pl.ds(0, _SUB)],
                                  row_v.at[gb], sg[gb]).wait()

            @pl.loop(0, _SUB // 16)
            def _(e16):
                vec = aidx_v[b, pl.ds(j * _SUB + e16 * 16, 16)]
                for u in range(16):
                    e = e16 * 16 + u
                    a = vec[u] - abase
                    owned = (a >= 0) & (a < _AQ)
                    t = jnp.where(owned, a, _AQ)
                    offs = t * 16 + lanes
                    plsc.addupdate_scatter(acc, [offs], row_v[gb, e, :])

        issue_idx(0, 0)
        issue_idx(1, 1)

        @pl.loop(0, nch, step=2)
        def _(chv):
            for b in (0, 1):
                cur = chv + b
                wait_idx(b)
                build_issue(cur, b, 0, 0)
                for j in range(1, _NSUB):
                    build_issue(cur, b, j, j % 2)
                    wait_process(b, j - 1, (j - 1) % 2)
                wait_process(b, _NSUB - 1, (_NSUB - 1) % 2)

                @pl.when(cur + 2 < nch)
                def _():
                    issue_idx(cur + 2, b)

        pltpu.sync_copy(acc, out_sum.at[q].at[f])

    return k(org_tab, org_idx, abn_idx, zeros_flat)


def _tc_lab_enh(lab_feats, lab_concept, w1t, w2t, b):
    blk = 1000
    grid = _NL // blk

    def body(x1, x2, w1, w2, bb, o):
        acc = jnp.dot(x1[...], w1[...], preferred_element_type=jnp.float32)
        acc = acc + jnp.dot(x2[...], w2[...],
                            preferred_element_type=jnp.float32)
        o[...] = jnp.maximum(acc + bb[...], 0.0)

    return pl.pallas_call(
        body,
        grid=(grid,),
        in_specs=[pl.BlockSpec((blk, _DL), lambda i: (i, 0)),
                  pl.BlockSpec((blk, _DC), lambda i: (i, 0)),
                  pl.BlockSpec((_DL, _DL), lambda i: (0, 0)),
                  pl.BlockSpec((_DC, _DL), lambda i: (0, 0)),
                  pl.BlockSpec((1, _DL), lambda i: (0, 0))],
        out_specs=pl.BlockSpec((blk, _DL), lambda i: (i, 0)),
        out_shape=jax.ShapeDtypeStruct((_NL, _DL), jnp.float32),
    )(lab_feats, lab_concept, w1t, w2t, b)


def _tc_rel_proj(rel_emb, dt):
    blk = 2000
    grid = _E1 // blk

    def body(x, d, o):
        o[...] = jnp.dot(x[...], d[...], preferred_element_type=jnp.float32)

    return pl.pallas_call(
        body,
        grid=(grid,),
        in_specs=[pl.BlockSpec((blk, _DC), lambda i: (i, 0)),
                  pl.BlockSpec((_DC, _DL), lambda i: (0, 0))],
        out_specs=pl.BlockSpec((blk, _DL), lambda i: (i, 0)),
        out_shape=jax.ShapeDtypeStruct((_E1, _DL), jnp.float32),
    )(rel_emb, dt)


def _tc_org(parts, cnts, cnts2, wot, wo2t, b1, b2):
    def body(pp, cp, cp2, wo, w2, bb1, bb2, o, o2):
        org_sum = (pp[0, :_NO, :] + pp[1, :_NO, :]
                   + pp[2, :_NO, :] + pp[3, :_NO, :])
        cnt = jnp.sum(cp[...], axis=0)[:_NO, None]
        org_agg = org_sum / jnp.maximum(cnt, 1.0)
        st = jnp.maximum(
            jnp.dot(org_agg, wo[...], preferred_element_type=jnp.float32)
            + bb1[...], 0.0)
        o[...] = jnp.dot(st, w2[...],
                         preferred_element_type=jnp.float32) + bb2[...]
        o2[...] = jnp.sum(cp2[...], axis=0)[:, None]

    return pl.pallas_call(
        body,
        out_shape=[jax.ShapeDtypeStruct((_NO, _DL), jnp.float32),
                   jax.ShapeDtypeStruct((_AQPAD * 4, 1), jnp.float32)],
    )(parts, cnts, cnts2, wot, wo2t, b1, b2)


def _tc_abn(abn_part, cnt_part, abn_feats, abn_concept, a1t, a2t, b):
    blk = 1000
    grid = _NA // blk
    per_q = _AQ // blk

    def body(ap, cp, ff, cc, w1, w2, bb, o):
        msg = ap[0] / jnp.maximum(cp[...], 1.0)
        x = ff[...] + msg
        o[...] = (jnp.dot(x, w1[...], preferred_element_type=jnp.float32)
                  + jnp.dot(cc[...], w2[...],
                            preferred_element_type=jnp.float32)
                  + bb[...])

    return pl.pallas_call(
        body,
        grid=(grid,),
        in_specs=[
            pl.BlockSpec((1, blk, _DL), lambda i: (i // per_q, i % per_q, 0)),
            pl.BlockSpec((blk, 1), lambda i: (i, 0)),
            pl.BlockSpec((blk, _DL), lambda i: (i, 0)),
            pl.BlockSpec((blk, _DC), lambda i: (i, 0)),
            pl.BlockSpec((_DL, _DL), lambda i: (0, 0)),
            pl.BlockSpec((_DC, _DL), lambda i: (0, 0)),
            pl.BlockSpec((1, _DL), lambda i: (0, 0)),
        ],
        out_specs=pl.BlockSpec((blk, _DL), lambda i: (i, 0)),
        out_shape=jax.ShapeDtypeStruct((_NA, _DL), jnp.float32),
    )(abn_part, cnt_part, abn_feats, abn_concept, a1t, a2t, b)


def kernel(lab_feats, abn_feats, lab_concept, abn_concept, lab_org_rel_emb,
           lab_org_lab_idx, lab_org_org_idx, o2a_abn_idx, o2a_org_idx,
           W_lab_w, W_lab_b, W_abn_w, W_abn_b, W_org_w, W_org_b,
           D_w, W_o2a_w, W_o2a_b):
    f32 = jnp.float32
    w1t = W_lab_w[:, :_DL].T
    w2t = W_lab_w[:, _DL:].T
    a1t = W_abn_w[:, :_DL].T
    a2t = W_abn_w[:, _DL:].T
    dt = D_w.T
    wot = W_org_w.T
    wo2t = W_o2a_w.T
    b_lab = W_lab_b.reshape(1, _DL)
    b_org = W_org_b.reshape(1, _DL)
    b_o2a = W_o2a_b.reshape(1, _DL)
    b_abn = W_abn_b.reshape(1, _DL)

    z_flat = jnp.zeros((_ACCW,), f32)

    c1, c2 = _sc_counts(lab_org_org_idx, o2a_abn_idx, z_flat)
    lab_enh = _tc_lab_enh(lab_feats, lab_concept, w1t, w2t, b_lab)
    rel_proj = _tc_rel_proj(lab_org_rel_emb, dt)
    parts = _sc_org_segsum(lab_enh.reshape(_NL * 8, 16),
                           rel_proj.reshape(_E1 * 8, 16),
                           lab_org_lab_idx, lab_org_org_idx, z_flat)
    parts = (parts.reshape(4, 8, _OPAD, 16).transpose(0, 2, 1, 3)
             .reshape(4, _OPAD, _DL))
    org_tab, cnt2 = _tc_org(parts, c1, c2, wot, wo2t, b_org, b_o2a)
    abn_part = _sc_abn_segsum(org_tab.reshape(_NO * 8, 16),
                              o2a_org_idx, o2a_abn_idx, z_flat)
    abn_part = (abn_part.reshape(4, 8, _AQPAD, 16).transpose(0, 2, 1, 3)
                .reshape(4, _AQPAD, _DL))
    return _tc_abn(abn_part, cnt2, abn_feats, abn_concept, a1t, a2t, b_abn)


# speedup vs baseline: 3.3287x; 1.3084x over previous
import functools

import jax
import jax.numpy as jnp
from jax import lax
from jax.experimental import pallas as pl
from jax.experimental.pallas import tpu as pltpu
from jax.experimental.pallas import tpu_sc as plsc

_NL, _NO, _NA = 50000, 5000, 20000
_E1, _E2 = 256000, 128000
_DL, _DC = 128, 256

_NC, _NS = 2, 16
_SUB = 128
_NSUB = 10
_CH = _SUB * _NSUB
_OPAD = 5120
_AQ = 5000
_AQPAD = 5120
_ACCW = _OPAD * 16


def _mesh():
    return plsc.VectorSubcoreMesh(core_axis_name="c", subcore_axis_name="s")


def _cidx(u):
    return jnp.full((16,), u, jnp.int32)


_SC_PARAMS = pltpu.CompilerParams(use_tc_tiling_on_sc=False,
                                  needs_layout_passes=False)


def _sc_counts(org_idx, abn_idx, zeros_flat):
    ep1 = _E1 // (_NC * _NS)
    ep2 = _E2 // (_NC * _NS)
    ch = 2000
    nb1 = ep1 // ch
    nb2 = ep2 // ch
    n2 = _AQPAD * 4

    @functools.partial(
        pl.kernel,
        out_type=[jax.ShapeDtypeStruct((32, _OPAD), jnp.float32),
                  jax.ShapeDtypeStruct((32, n2), jnp.float32)],
        mesh=_mesh(),
        scratch_types=[pltpu.VMEM((ch,), jnp.int32),
                       pltpu.VMEM((_OPAD,), jnp.float32),
                       pltpu.VMEM((n2,), jnp.float32)],
        compiler_params=_SC_PARAMS,
    )
    def k(oidx_hbm, aidx_hbm, zs_hbm, out_c1, out_c2, idx_v, acc1, acc2):
        c = lax.axis_index("c")
        s = lax.axis_index("s")
        wid = c * _NS + s
        lanes = lax.iota(jnp.int32, 16)
        mask0 = lanes == 0
        ones = jnp.ones((16,), jnp.float32)
        pltpu.sync_copy(zs_hbm.at[pl.ds(0, _OPAD)], acc1)
        pltpu.sync_copy(zs_hbm.at[pl.ds(0, n2)], acc2)

        def count_into(acc, idx_hbm, ep, nb):
            @pl.loop(0, nb)
            def _(i):
                pltpu.sync_copy(idx_hbm.at[pl.ds(wid * ep + i * ch, ch)],
                                idx_v)

                @pl.loop(0, ch // 16)
                def _(e16):
                    vec = idx_v[pl.ds(e16 * 16, 16)]
                    for u in range(16):
                        offs = vec.at[_cidx(u)].get(
                            mode="promise_in_bounds")
                        plsc.addupdate_scatter(acc, [offs], ones, mask=mask0)

        count_into(acc1, oidx_hbm, ep1, nb1)
        count_into(acc2, aidx_hbm, ep2, nb2)
        pltpu.sync_copy(acc1, out_c1.at[wid])
        pltpu.sync_copy(acc2, out_c2.at[wid])

    return k(org_idx, abn_idx, zeros_flat)


def _sc_org_segsum(lab_enh, rel_proj, lab_idx, org_idx, zeros_flat):
    epg = _E1 // 4
    nch = epg // _CH

    @functools.partial(
        pl.kernel,
        out_type=jax.ShapeDtypeStruct((4, 8, _ACCW), jnp.float32),
        mesh=_mesh(),
        scratch_types=[pltpu.VMEM((2, _CH), jnp.int32),
                       pltpu.VMEM((2, _CH), jnp.int32),
                       pltpu.VMEM((2, _SUB), jnp.int32),
                       pltpu.VMEM((2, _SUB), jnp.int32),
                       pltpu.VMEM((2, _SUB, 16), jnp.float32),
                       pltpu.VMEM((2, _SUB, 16), jnp.float32),
                       pltpu.VMEM((_ACCW,), jnp.float32),
                       pltpu.SemaphoreType.DMA,
                       pltpu.SemaphoreType.DMA,
                       pltpu.SemaphoreType.DMA,
                       pltpu.SemaphoreType.DMA],
        compiler_params=_SC_PARAMS,
    )
    def k(tab_hbm, rel_hbm, lidx_hbm, oidx_hbm, zs_hbm, out_sum,
          lidx_v, oidx_v, gi_v, ri_v, lab_v, rel_v, acc,
          si0, si1, sg0, sg1):
        c = lax.axis_index("c")
        s = lax.axis_index("s")
        wid = c * _NS + s
        f = wid % 8
        g = wid // 8
        lanes = lax.iota(jnp.int32, 16)
        lanes8 = lanes * 8
        pltpu.sync_copy(zs_hbm, acc)
        base = g * epg
        si = (si0, si1)
        sg = (sg0, sg1)

        def issue_idx(chv, b):
            off = base + chv * _CH
            pltpu.async_copy(lidx_hbm.at[pl.ds(off, _CH)], lidx_v.at[b],
                             si[b])
            pltpu.async_copy(oidx_hbm.at[pl.ds(off, _CH)], oidx_v.at[b],
                             si[b])

        def wait_idx(b):
            pltpu.make_async_copy(lidx_hbm.at[pl.ds(0, _CH)],
                                  lidx_v.at[b], si[b]).wait()
            pltpu.make_async_copy(oidx_hbm.at[pl.ds(0, _CH)],
                                  oidx_v.at[b], si[b]).wait()

        def build_issue(chv, b, j, gb):
            off = base + chv * _CH + j * _SUB

            @pl.loop(0, _SUB // 16)
            def _(c16):
                lvec = lidx_v[b, pl.ds(j * _SUB + c16 * 16, 16)]
                gi_v[gb, pl.ds(c16 * 16, 16)] = lvec * 8 + f
                ri_v[gb, pl.ds(c16 * 16, 16)] = ((off + c16 * 16) * 8 + f
                                                 + lanes8)

            pltpu.async_copy(tab_hbm.at[gi_v.at[gb]], lab_v.at[gb], sg[gb])
            pltpu.async_copy(rel_hbm.at[ri_v.at[gb]], rel_v.at[gb], sg[gb])

        def wait_process(b, j, gb):
            pltpu.make_async_copy(tab_hbm.at[pl.ds(0, _SUB)],
                                  lab_v.at[gb], sg[gb]).wait()
            pltpu.make_async_copy(rel_hbm.at[pl.ds(0, _SUB)],
                                  rel_v.at[gb], sg[gb]).wait()

            @pl.loop(0, _SUB // 16)
            def _(e16):
                vec = oidx_v[b, pl.ds(j * _SUB + e16 * 16, 16)]
                base16 = vec * 16
                es = [e16 * 16 + u for u in range(16)]
                labs = [lab_v[gb, e, :] for e in es]
                rels = [rel_v[gb, e, :] for e in es]
                offs = [base16.at[_cidx(u)].get(
                    mode="promise_in_bounds") + lanes for u in range(16)]
                vals = [a + r for a, r in zip(labs, rels)]
                for u in range(16):
                    plsc.addupdate_scatter(acc, [offs[u]], vals[u])

        issue_idx(0, 0)
        issue_idx(1, 1)

        @pl.loop(0, nch, step=2)
        def _(chv):
            for b in (0, 1):
                cur = chv + b
                wait_idx(b)
                build_issue(cur, b, 0, 0)
                for j in range(1, _NSUB):
                    build_issue(cur, b, j, j % 2)
                    wait_process(b, j - 1, (j - 1) % 2)
                wait_process(b, _NSUB - 1, (_NSUB - 1) % 2)

                @pl.when(cur + 2 < nch)
                def _():
                    issue_idx(cur + 2, b)

        pltpu.sync_copy(acc, out_sum.at[g].at[f])

    return k(lab_enh, rel_proj, lab_idx, org_idx, zeros_flat)


def _sc_abn_segsum(org_tab, org_idx, abn_idx, zeros_flat):
    nch = _E2 // _CH

    @functools.partial(
        pl.kernel,
        out_type=jax.ShapeDtypeStruct((4, 8, _ACCW), jnp.float32),
        mesh=_mesh(),
        scratch_types=[pltpu.VMEM((2, _CH), jnp.int32),
                       pltpu.VMEM((2, _CH), jnp.int32),
                       pltpu.VMEM((2, _SUB), jnp.int32),
                       pltpu.VMEM((2, _SUB, 16), jnp.float32),
                       pltpu.VMEM((_ACCW,), jnp.float32),
                       pltpu.SemaphoreType.DMA,
                       pltpu.SemaphoreType.DMA,
                       pltpu.SemaphoreType.DMA,
                       pltpu.SemaphoreType.DMA],
        compiler_params=_SC_PARAMS,
    )
    def k(tab_hbm, oidx_hbm, aidx_hbm, zs_hbm, out_sum,
          oidx_v, aidx_v, gi_v, row_v, acc, si0, si1, sg0, sg1):
        c = lax.axis_index("c")
        s = lax.axis_index("s")
        wid = c * _NS + s
        f = wid % 8
        q = wid // 8
        abase = q * _AQ
        lanes = lax.iota(jnp.int32, 16)
        pltpu.sync_copy(zs_hbm, acc)
        si = (si0, si1)
        sg = (sg0, sg1)

        def issue_idx(chv, b):
            off = chv * _CH
            pltpu.async_copy(oidx_hbm.at[pl.ds(off, _CH)], oidx_v.at[b],
                             si[b])
            pltpu.async_copy(aidx_hbm.at[pl.ds(off, _CH)], aidx_v.at[b],
                             si[b])

        def wait_idx(b):
            pltpu.make_async_copy(oidx_hbm.at[pl.ds(0, _CH)],
                                  oidx_v.at[b], si[b]).wait()
            pltpu.make_async_copy(aidx_hbm.at[pl.ds(0, _CH)],
                                  aidx_v.at[b], si[b]).wait()

        def build_issue(chv, b, j, gb):
            @pl.loop(0, _SUB // 16)
            def _(c16):
                ovec = oidx_v[b, pl.ds(j * _SUB + c16 * 16, 16)]
                gi_v[gb, pl.ds(c16 * 16, 16)] = ovec * 8 + f

            pltpu.async_copy(tab_hbm.at[gi_v.at[gb]], row_v.at[gb], sg[gb])

        def wait_process(b, j, gb):
            pltpu.make_async_copy(tab_hbm.at[pl.ds(0, _SUB)],
                                  row_v.at[gb], sg[gb]).wait()

            @pl.loop(0, _SUB // 16)
            def _(e16):
                vec = aidx_v[b, pl.ds(j * _SUB + e16 * 16, 16)]
                av = vec - abase
                owned = (av >= 0) & (av < _AQ)
                base16 = jnp.where(owned, av, _AQ) * 16
                es = [e16 * 16 + u for u in range(16)]
                rows = [row_v[gb, e, :] for e in es]
                offs = [base16.at[_cidx(u)].get(
                    mode="promise_in_bounds") + lanes for u in range(16)]
                for u in range(16):
                    plsc.addupdate_scatter(acc, [offs[u]], rows[u])

        issue_idx(0, 0)
        issue_idx(1, 1)

        @pl.loop(0, nch, step=2)
        def _(chv):
            for b in (0, 1):
                cur = chv + b
                wait_idx(b)
                build_issue(cur, b, 0, 0)
                for j in range(1, _NSUB):
                    build_issue(cur, b, j, j % 2)
                    wait_process(b, j - 1, (j - 1) % 2)
                wait_process(b, _NSUB - 1, (_NSUB - 1) % 2)

                @pl.when(cur + 2 < nch)
                def _():
                    issue_idx(cur + 2, b)

        pltpu.sync_copy(acc, out_sum.at[q].at[f])

    return k(org_tab, org_idx, abn_idx, zeros_flat)


def _tc_lab_enh(lab_feats, lab_concept, w1t, w2t, b):
    blk = 1000
    grid = _NL // blk

    def body(x1, x2, w1, w2, bb, o):
        bf = jnp.bfloat16
        acc = jnp.dot(x1[...].astype(bf), w1[...].astype(bf),
                      preferred_element_type=jnp.float32)
        acc = acc + jnp.dot(x2[...].astype(bf), w2[...].astype(bf),
                            preferred_element_type=jnp.float32)
        o[...] = jnp.maximum(acc + bb[...], 0.0)

    return pl.pallas_call(
        body,
        grid=(grid,),
        in_specs=[pl.BlockSpec((blk, _DL), lambda i: (i, 0)),
                  pl.BlockSpec((blk, _DC), lambda i: (i, 0)),
                  pl.BlockSpec((_DL, _DL), lambda i: (0, 0)),
                  pl.BlockSpec((_DC, _DL), lambda i: (0, 0)),
                  pl.BlockSpec((1, _DL), lambda i: (0, 0))],
        out_specs=pl.BlockSpec((blk, _DL), lambda i: (i, 0)),
        out_shape=jax.ShapeDtypeStruct((_NL, _DL), jnp.float32),
    )(lab_feats, lab_concept, w1t, w2t, b)


def _tc_rel_proj(rel_emb, dt):
    blk = 2000
    grid = _E1 // blk

    def body(x, d, o):
        bf = jnp.bfloat16
        o[...] = jnp.dot(x[...].astype(bf), d[...].astype(bf),
                        preferred_element_type=jnp.float32)

    return pl.pallas_call(
        body,
        grid=(grid,),
        in_specs=[pl.BlockSpec((blk, _DC), lambda i: (i, 0)),
                  pl.BlockSpec((_DC, _DL), lambda i: (0, 0))],
        out_specs=pl.BlockSpec((blk, _DL), lambda i: (i, 0)),
        out_shape=jax.ShapeDtypeStruct((_E1, _DL), jnp.float32),
    )(rel_emb, dt)


def _tc_org(parts, cnts, cnts2, wot, wo2t, b1, b2):
    def body(pp, cp, cp2, wo, w2, bb1, bb2, o, o2):
        org_sum = (pp[0, :_NO, :] + pp[1, :_NO, :]
                   + pp[2, :_NO, :] + pp[3, :_NO, :])
        cnt = jnp.sum(cp[...], axis=0)[:_NO, None]
        org_agg = org_sum / jnp.maximum(cnt, 1.0)
        st = jnp.maximum(
            jnp.dot(org_agg, wo[...], preferred_element_type=jnp.float32)
            + bb1[...], 0.0)
        o[...] = jnp.dot(st, w2[...],
                         preferred_element_type=jnp.float32) + bb2[...]
        o2[...] = jnp.sum(cp2[...], axis=0)[:, None]

    return pl.pallas_call(
        body,
        out_shape=[jax.ShapeDtypeStruct((_NO, _DL), jnp.float32),
                   jax.ShapeDtypeStruct((_AQPAD * 4, 1), jnp.float32)],
    )(parts, cnts, cnts2, wot, wo2t, b1, b2)


def _tc_abn(abn_part, cnt_part, abn_feats, abn_concept, a1t, a2t, b):
    blk = 1000
    grid = _NA // blk
    per_q = _AQ // blk

    def body(ap, cp, ff, cc, w1, w2, bb, o):
        msg = ap[0] / jnp.maximum(cp[...], 1.0)
        bf = jnp.bfloat16
        x = (ff[...] + msg).astype(bf)
        o[...] = (jnp.dot(x, w1[...].astype(bf),
                          preferred_element_type=jnp.float32)
                  + jnp.dot(cc[...].astype(bf), w2[...].astype(bf),
                            preferred_element_type=jnp.float32)
                  + bb[...])

    return pl.pallas_call(
        body,
        grid=(grid,),
        in_specs=[
            pl.BlockSpec((1, blk, _DL), lambda i: (i // per_q, i % per_q, 0)),
            pl.BlockSpec((blk, 1), lambda i: (i, 0)),
            pl.BlockSpec((blk, _DL), lambda i: (i, 0)),
            pl.BlockSpec((blk, _DC), lambda i: (i, 0)),
            pl.BlockSpec((_DL, _DL), lambda i: (0, 0)),
            pl.BlockSpec((_DC, _DL), lambda i: (0, 0)),
            pl.BlockSpec((1, _DL), lambda i: (0, 0)),
        ],
        out_specs=pl.BlockSpec((blk, _DL), lambda i: (i, 0)),
        out_shape=jax.ShapeDtypeStruct((_NA, _DL), jnp.float32),
    )(abn_part, cnt_part, abn_feats, abn_concept, a1t, a2t, b)


def kernel(lab_feats, abn_feats, lab_concept, abn_concept, lab_org_rel_emb,
           lab_org_lab_idx, lab_org_org_idx, o2a_abn_idx, o2a_org_idx,
           W_lab_w, W_lab_b, W_abn_w, W_abn_b, W_org_w, W_org_b,
           D_w, W_o2a_w, W_o2a_b):
    f32 = jnp.float32
    w1t = W_lab_w[:, :_DL].T
    w2t = W_lab_w[:, _DL:].T
    a1t = W_abn_w[:, :_DL].T
    a2t = W_abn_w[:, _DL:].T
    dt = D_w.T
    wot = W_org_w.T
    wo2t = W_o2a_w.T
    b_lab = W_lab_b.reshape(1, _DL)
    b_org = W_org_b.reshape(1, _DL)
    b_o2a = W_o2a_b.reshape(1, _DL)
    b_abn = W_abn_b.reshape(1, _DL)

    z_flat = jnp.zeros((_ACCW,), f32)

    c1, c2 = _sc_counts(lab_org_org_idx, o2a_abn_idx, z_flat)
    lab_enh = _tc_lab_enh(lab_feats, lab_concept, w1t, w2t, b_lab)
    rel_proj = _tc_rel_proj(lab_org_rel_emb, dt)
    parts = _sc_org_segsum(lab_enh.reshape(_NL * 8, 16),
                           rel_proj.reshape(_E1 * 8, 16),
                           lab_org_lab_idx, lab_org_org_idx, z_flat)
    parts = (parts.reshape(4, 8, _OPAD, 16).transpose(0, 2, 1, 3)
             .reshape(4, _OPAD, _DL))
    org_tab, cnt2 = _tc_org(parts, c1, c2, wot, wo2t, b_org, b_o2a)
    abn_part = _sc_abn_segsum(org_tab.reshape(_NO * 8, 16),
                              o2a_org_idx, o2a_abn_idx, z_flat)
    abn_part = (abn_part.reshape(4, 8, _AQPAD, 16).transpose(0, 2, 1, 3)
                .reshape(4, _AQPAD, _DL))
    return _tc_abn(abn_part, cnt2, abn_feats, abn_concept, a1t, a2t, b_abn)


# speedup vs baseline: 4.2640x; 1.2810x over previous
import functools

import jax
import jax.numpy as jnp
from jax import lax
from jax.experimental import pallas as pl
from jax.experimental.pallas import tpu as pltpu
from jax.experimental.pallas import tpu_sc as plsc

_NL, _NO, _NA = 50000, 5000, 20000
_E1, _E2 = 256000, 128000
_DL, _DC = 128, 256

_NC, _NS = 2, 16
_SUB = 128
_NSUB = 10
_CH = _SUB * _NSUB
_OPAD = 5120
_AQ = 5000
_AQPAD = 5008
_ACCW = _OPAD * 16
_ACCWA = _AQPAD * 16


def _mesh():
    return plsc.VectorSubcoreMesh(core_axis_name="c", subcore_axis_name="s")


def _cidx(u):
    return jnp.full((16,), u, jnp.int32)


_SC_PARAMS = pltpu.CompilerParams(use_tc_tiling_on_sc=False,
                                  needs_layout_passes=False)


def _sc_counts(org_idx, abn_idx, zeros_flat):
    ep1 = _E1 // (_NC * _NS)
    ep2 = _E2 // (_NC * _NS)
    ch = 2000
    nb1 = ep1 // ch
    nb2 = ep2 // ch
    n2 = _AQPAD * 4

    @functools.partial(
        pl.kernel,
        out_type=[jax.ShapeDtypeStruct((32, _OPAD), jnp.float32),
                  jax.ShapeDtypeStruct((32, n2), jnp.float32)],
        mesh=_mesh(),
        scratch_types=[pltpu.VMEM((ch,), jnp.int32),
                       pltpu.VMEM((_OPAD,), jnp.float32),
                       pltpu.VMEM((n2,), jnp.float32)],
        compiler_params=_SC_PARAMS,
    )
    def k(oidx_hbm, aidx_hbm, zs_hbm, out_c1, out_c2, idx_v, acc1, acc2):
        c = lax.axis_index("c")
        s = lax.axis_index("s")
        wid = c * _NS + s
        lanes = lax.iota(jnp.int32, 16)
        mask0 = lanes == 0
        ones = jnp.ones((16,), jnp.float32)
        pltpu.sync_copy(zs_hbm.at[pl.ds(0, _OPAD)], acc1)
        pltpu.sync_copy(zs_hbm.at[pl.ds(0, n2)], acc2)

        def count_into(acc, idx_hbm, ep, nb):
            @pl.loop(0, nb)
            def _(i):
                pltpu.sync_copy(idx_hbm.at[pl.ds(wid * ep + i * ch, ch)],
                                idx_v)

                @pl.loop(0, ch // 16)
                def _(e16):
                    vec = idx_v[pl.ds(e16 * 16, 16)]
                    for u in range(16):
                        offs = vec.at[_cidx(u)].get(
                            mode="promise_in_bounds")
                        plsc.addupdate_scatter(acc, [offs], ones, mask=mask0)

        count_into(acc1, oidx_hbm, ep1, nb1)
        count_into(acc2, aidx_hbm, ep2, nb2)
        pltpu.sync_copy(acc1, out_c1.at[wid])
        pltpu.sync_copy(acc2, out_c2.at[wid])

    return k(org_idx, abn_idx, zeros_flat)


def _sc_org_segsum(lab_enh, rel_proj, lab_idx, org_idx, zeros_flat):
    epg = _E1 // 4
    nch = epg // _CH

    @functools.partial(
        pl.kernel,
        out_type=jax.ShapeDtypeStruct((4, 8, _ACCW), jnp.float32),
        mesh=_mesh(),
        scratch_types=[pltpu.VMEM((2, _CH), jnp.int32),
                       pltpu.VMEM((2, _CH), jnp.int32),
                       pltpu.VMEM((2, _SUB), jnp.int32),
                       pltpu.VMEM((2, _SUB), jnp.int32),
                       pltpu.VMEM((2, _SUB, 16), jnp.float32),
                       pltpu.VMEM((2, _SUB, 16), jnp.float32),
                       pltpu.VMEM((_ACCW,), jnp.float32),
                       pltpu.SemaphoreType.DMA,
                       pltpu.SemaphoreType.DMA,
                       pltpu.SemaphoreType.DMA,
                       pltpu.SemaphoreType.DMA],
        compiler_params=_SC_PARAMS,
    )
    def k(tab_hbm, rel_hbm, lidx_hbm, oidx_hbm, zs_hbm, out_sum,
          lidx_v, oidx_v, gi_v, ri_v, lab_v, rel_v, acc,
          si0, si1, sg0, sg1):
        c = lax.axis_index("c")
        s = lax.axis_index("s")
        wid = c * _NS + s
        f = wid % 8
        g = wid // 8
        lanes = lax.iota(jnp.int32, 16)
        lanes8 = lanes * 8
        pltpu.sync_copy(zs_hbm, acc)
        base = g * epg
        si = (si0, si1)
        sg = (sg0, sg1)

        def issue_idx(chv, b):
            off = base + chv * _CH
            pltpu.async_copy(lidx_hbm.at[pl.ds(off, _CH)], lidx_v.at[b],
                             si[b])
            pltpu.async_copy(oidx_hbm.at[pl.ds(off, _CH)], oidx_v.at[b],
                             si[b])

        def wait_idx(b):
            pltpu.make_async_copy(lidx_hbm.at[pl.ds(0, _CH)],
                                  lidx_v.at[b], si[b]).wait()
            pltpu.make_async_copy(oidx_hbm.at[pl.ds(0, _CH)],
                                  oidx_v.at[b], si[b]).wait()

        def build_issue(chv, b, j, gb):
            off = base + chv * _CH + j * _SUB

            @pl.loop(0, _SUB // 16)
            def _(c16):
                lvec = lidx_v[b, pl.ds(j * _SUB + c16 * 16, 16)]
                gi_v[gb, pl.ds(c16 * 16, 16)] = lvec * 8 + f
                ri_v[gb, pl.ds(c16 * 16, 16)] = ((off + c16 * 16) * 8 + f
                                                 + lanes8)

            pltpu.async_copy(tab_hbm.at[gi_v.at[gb]], lab_v.at[gb], sg[gb])
            pltpu.async_copy(rel_hbm.at[ri_v.at[gb]], rel_v.at[gb], sg[gb])

        def wait_process(b, j, gb):
            pltpu.make_async_copy(tab_hbm.at[pl.ds(0, _SUB)],
                                  lab_v.at[gb], sg[gb]).wait()
            pltpu.make_async_copy(rel_hbm.at[pl.ds(0, _SUB)],
                                  rel_v.at[gb], sg[gb]).wait()

            @pl.loop(0, _SUB // 16)
            def _(e16):
                vec = oidx_v[b, pl.ds(j * _SUB + e16 * 16, 16)]
                base16 = vec * 16
                es = [e16 * 16 + u for u in range(16)]
                labs = [lab_v[gb, e, :] for e in es]
                rels = [rel_v[gb, e, :] for e in es]
                offs = [base16.at[_cidx(u)].get(
                    mode="promise_in_bounds") + lanes for u in range(16)]
                vals = [a + r for a, r in zip(labs, rels)]
                for u in range(16):
                    plsc.addupdate_scatter(acc, [offs[u]], vals[u])

        issue_idx(0, 0)
        issue_idx(1, 1)

        @pl.loop(0, nch, step=2)
        def _(chv):
            for b in (0, 1):
                cur = chv + b
                wait_idx(b)
                build_issue(cur, b, 0, 0)
                for j in range(1, _NSUB):
                    build_issue(cur, b, j, j % 2)
                    wait_process(b, j - 1, (j - 1) % 2)
                wait_process(b, _NSUB - 1, (_NSUB - 1) % 2)

                @pl.when(cur + 2 < nch)
                def _():
                    issue_idx(cur + 2, b)

        pltpu.sync_copy(acc, out_sum.at[g].at[f])

    return k(lab_enh, rel_proj, lab_idx, org_idx, zeros_flat)


def _sc_abn_segsum(org_tab, org_idx, abn_idx, zeros_flat):
    nch = _E2 // _CH

    @functools.partial(
        pl.kernel,
        out_type=jax.ShapeDtypeStruct((4, 8, _ACCWA), jnp.float32),
        mesh=_mesh(),
        scratch_types=[pltpu.VMEM((2, _CH), jnp.int32),
                       pltpu.VMEM((2, _CH), jnp.int32),
                       pltpu.VMEM((2, _SUB), jnp.int32),
                       pltpu.VMEM((2, _SUB, 16), jnp.float32),
                       pltpu.VMEM((_ACCWA,), jnp.float32),
                       pltpu.VMEM_SHARED((_NO * 8, 16), jnp.float32),
                       pltpu.SemaphoreType.DMA,
                       pltpu.SemaphoreType.DMA,
                       pltpu.SemaphoreType.DMA,
                       pltpu.SemaphoreType.DMA],
        compiler_params=_SC_PARAMS,
    )
    def k(tab_hbm, oidx_hbm, aidx_hbm, zs_hbm, out_sum,
          oidx_v, aidx_v, gi_v, row_v, acc, stab, si0, si1, sg0, sg1):
        c = lax.axis_index("c")
        s = lax.axis_index("s")
        wid = c * _NS + s
        f = wid % 8
        q = wid // 8
        abase = q * _AQ
        lanes = lax.iota(jnp.int32, 16)
        pltpu.sync_copy(zs_hbm.at[pl.ds(0, _ACCWA)], acc)
        trows = _NO * 8 // _NS
        pltpu.sync_copy(tab_hbm.at[pl.ds(s * trows, trows)],
                        stab.at[pl.ds(s * trows, trows)])
        plsc.subcore_barrier()
        si = (si0, si1)
        sg = (sg0, sg1)

        def issue_idx(chv, b):
            off = chv * _CH
            pltpu.async_copy(oidx_hbm.at[pl.ds(off, _CH)], oidx_v.at[b],
                             si[b])
            pltpu.async_copy(aidx_hbm.at[pl.ds(off, _CH)], aidx_v.at[b],
                             si[b])

        def wait_idx(b):
            pltpu.make_async_copy(oidx_hbm.at[pl.ds(0, _CH)],
                                  oidx_v.at[b], si[b]).wait()
            pltpu.make_async_copy(aidx_hbm.at[pl.ds(0, _CH)],
                                  aidx_v.at[b], si[b]).wait()

        def build_issue(chv, b, j, gb):
            @pl.loop(0, _SUB // 16)
            def _(c16):
                ovec = oidx_v[b, pl.ds(j * _SUB + c16 * 16, 16)]
                gi_v[gb, pl.ds(c16 * 16, 16)] = ovec * 8 + f

            pltpu.async_copy(stab.at[gi_v.at[gb]], row_v.at[gb], sg[gb])

        def wait_process(b, j, gb):
            pltpu.make_async_copy(tab_hbm.at[pl.ds(0, _SUB)],
                                  row_v.at[gb], sg[gb]).wait()

            @pl.loop(0, _SUB // 16)
            def _(e16):
                vec = aidx_v[b, pl.ds(j * _SUB + e16 * 16, 16)]
                av = vec - abase
                owned = (av >= 0) & (av < _AQ)
                base16 = jnp.where(owned, av, _AQ) * 16
                es = [e16 * 16 + u for u in range(16)]
                rows = [row_v[gb, e, :] for e in es]
                offs = [base16.at[_cidx(u)].get(
                    mode="promise_in_bounds") + lanes for u in range(16)]
                for u in range(16):
                    plsc.addupdate_scatter(acc, [offs[u]], rows[u])

        issue_idx(0, 0)
        issue_idx(1, 1)

        @pl.loop(0, nch, step=2)
        def _(chv):
            for b in (0, 1):
                cur = chv + b
                wait_idx(b)
                build_issue(cur, b, 0, 0)
                for j in range(1, _NSUB):
                    build_issue(cur, b, j, j % 2)
                    wait_process(b, j - 1, (j - 1) % 2)
                wait_process(b, _NSUB - 1, (_NSUB - 1) % 2)

                @pl.when(cur + 2 < nch)
                def _():
                    issue_idx(cur + 2, b)

        pltpu.sync_copy(acc, out_sum.at[q].at[f])

    return k(org_tab, org_idx, abn_idx, zeros_flat)


def _tc_lab_enh(lab_feats, lab_concept, w1t, w2t, b):
    blk = 1000
    grid = _NL // blk

    def body(x1, x2, w1, w2, bb, o):
        bf = jnp.bfloat16
        acc = jnp.dot(x1[...].astype(bf), w1[...].astype(bf),
                      preferred_element_type=jnp.float32)
        acc = acc + jnp.dot(x2[...].astype(bf), w2[...].astype(bf),
                            preferred_element_type=jnp.float32)
        o[...] = jnp.maximum(acc + bb[...], 0.0)

    return pl.pallas_call(
        body,
        grid=(grid,),
        in_specs=[pl.BlockSpec((blk, _DL), lambda i: (i, 0)),
                  pl.BlockSpec((blk, _DC), lambda i: (i, 0)),
                  pl.BlockSpec((_DL, _DL), lambda i: (0, 0)),
                  pl.BlockSpec((_DC, _DL), lambda i: (0, 0)),
                  pl.BlockSpec((1, _DL), lambda i: (0, 0))],
        out_specs=pl.BlockSpec((blk, _DL), lambda i: (i, 0)),
        out_shape=jax.ShapeDtypeStruct((_NL, _DL), jnp.float32),
    )(lab_feats, lab_concept, w1t, w2t, b)


def _tc_rel_proj(rel_emb, dt):
    blk = 2000
    grid = _E1 // blk

    def body(x, d, o):
        bf = jnp.bfloat16
        o[...] = jnp.dot(x[...].astype(bf), d[...].astype(bf),
                        preferred_element_type=jnp.float32)

    return pl.pallas_call(
        body,
        grid=(grid,),
        in_specs=[pl.BlockSpec((blk, _DC), lambda i: (i, 0)),
                  pl.BlockSpec((_DC, _DL), lambda i: (0, 0))],
        out_specs=pl.BlockSpec((blk, _DL), lambda i: (i, 0)),
        out_shape=jax.ShapeDtypeStruct((_E1, _DL), jnp.float32),
    )(rel_emb, dt)


def _tc_org(parts, cnts, cnts2, wot, wo2t, b1, b2):
    def body(pp, cp, cp2, wo, w2, bb1, bb2, o, o2):
        org_sum = (pp[0, :_NO, :] + pp[1, :_NO, :]
                   + pp[2, :_NO, :] + pp[3, :_NO, :])
        cnt = jnp.sum(cp[...], axis=0)[:_NO, None]
        org_agg = org_sum / jnp.maximum(cnt, 1.0)
        st = jnp.maximum(
            jnp.dot(org_agg, wo[...], preferred_element_type=jnp.float32)
            + bb1[...], 0.0)
        o[...] = jnp.dot(st, w2[...],
                         preferred_element_type=jnp.float32) + bb2[...]
        o2[...] = jnp.sum(cp2[...], axis=0)[:, None]

    return pl.pallas_call(
        body,
        out_shape=[jax.ShapeDtypeStruct((_NO, _DL), jnp.float32),
                   jax.ShapeDtypeStruct((_AQPAD * 4, 1), jnp.float32)],
    )(parts, cnts, cnts2, wot, wo2t, b1, b2)


def _tc_abn(abn_part, cnt_part, abn_feats, abn_concept, a1t, a2t, b):
    blk = 1000
    grid = _NA // blk
    per_q = _AQ // blk

    def body(ap, cp, ff, cc, w1, w2, bb, o):
        msg = ap[0] / jnp.maximum(cp[...], 1.0)
        bf = jnp.bfloat16
        x = (ff[...] + msg).astype(bf)
        o[...] = (jnp.dot(x, w1[...].astype(bf),
                          preferred_element_type=jnp.float32)
                  + jnp.dot(cc[...].astype(bf), w2[...].astype(bf),
                            preferred_element_type=jnp.float32)
                  + bb[...])

    return pl.pallas_call(
        body,
        grid=(grid,),
        in_specs=[
            pl.BlockSpec((1, blk, _DL), lambda i: (i // per_q, i % per_q, 0)),
            pl.BlockSpec((blk, 1), lambda i: (i, 0)),
            pl.BlockSpec((blk, _DL), lambda i: (i, 0)),
            pl.BlockSpec((blk, _DC), lambda i: (i, 0)),
            pl.BlockSpec((_DL, _DL), lambda i: (0, 0)),
            pl.BlockSpec((_DC, _DL), lambda i: (0, 0)),
            pl.BlockSpec((1, _DL), lambda i: (0, 0)),
        ],
        out_specs=pl.BlockSpec((blk, _DL), lambda i: (i, 0)),
        out_shape=jax.ShapeDtypeStruct((_NA, _DL), jnp.float32),
    )(abn_part, cnt_part, abn_feats, abn_concept, a1t, a2t, b)


def kernel(lab_feats, abn_feats, lab_concept, abn_concept, lab_org_rel_emb,
           lab_org_lab_idx, lab_org_org_idx, o2a_abn_idx, o2a_org_idx,
           W_lab_w, W_lab_b, W_abn_w, W_abn_b, W_org_w, W_org_b,
           D_w, W_o2a_w, W_o2a_b):
    f32 = jnp.float32
    w1t = W_lab_w[:, :_DL].T
    w2t = W_lab_w[:, _DL:].T
    a1t = W_abn_w[:, :_DL].T
    a2t = W_abn_w[:, _DL:].T
    dt = D_w.T
    wot = W_org_w.T
    wo2t = W_o2a_w.T
    b_lab = W_lab_b.reshape(1, _DL)
    b_org = W_org_b.reshape(1, _DL)
    b_o2a = W_o2a_b.reshape(1, _DL)
    b_abn = W_abn_b.reshape(1, _DL)

    z_flat = jnp.zeros((_ACCW,), f32)

    c1, c2 = _sc_counts(lab_org_org_idx, o2a_abn_idx, z_flat)
    lab_enh = _tc_lab_enh(lab_feats, lab_concept, w1t, w2t, b_lab)
    rel_proj = _tc_rel_proj(lab_org_rel_emb, dt)
    parts = _sc_org_segsum(lab_enh.reshape(_NL * 8, 16),
                           rel_proj.reshape(_E1 * 8, 16),
                           lab_org_lab_idx, lab_org_org_idx, z_flat)
    parts = (parts.reshape(4, 8, _OPAD, 16).transpose(0, 2, 1, 3)
             .reshape(4, _OPAD, _DL))
    org_tab, cnt2 = _tc_org(parts, c1, c2, wot, wo2t, b_org, b_o2a)
    abn_part = _sc_abn_segsum(org_tab.reshape(_NO * 8, 16),
                              o2a_org_idx, o2a_abn_idx, z_flat)
    abn_part = (abn_part.reshape(4, 8, _AQPAD, 16).transpose(0, 2, 1, 3)
                .reshape(4, _AQPAD, _DL))
    return _tc_abn(abn_part, cnt2, abn_feats, abn_concept, a1t, a2t, b_abn)


# speedup vs baseline: 4.7488x; 1.1137x over previous
import functools

import jax
import jax.numpy as jnp
from jax import lax
from jax.experimental import pallas as pl
from jax.experimental.pallas import tpu as pltpu
from jax.experimental.pallas import tpu_sc as plsc

_NL, _NO, _NA = 50000, 5000, 20000
_E1, _E2 = 256000, 128000
_DL, _DC = 128, 256

_NC, _NS = 2, 16
_SUB = 128
_NSUB = 10
_CH = _SUB * _NSUB
_OPAD = 5120
_AQ = 5000
_AQPAD = 5008
_ACCW = _OPAD * 16
_ACCWA = _AQPAD * 16


def _mesh():
    return plsc.VectorSubcoreMesh(core_axis_name="c", subcore_axis_name="s")


def _cidx(u):
    return jnp.full((16,), u, jnp.int32)


_SC_PARAMS = pltpu.CompilerParams(use_tc_tiling_on_sc=False,
                                  needs_layout_passes=False)


def _sc_counts(org_idx, abn_idx, zeros_flat):
    ep1 = _E1 // (_NC * _NS)
    ep2 = _E2 // (_NC * _NS)
    ch = 2000
    nb1 = ep1 // ch
    nb2 = ep2 // ch
    n2 = _AQPAD * 4

    @functools.partial(
        pl.kernel,
        out_type=[jax.ShapeDtypeStruct((32, _OPAD), jnp.float32),
                  jax.ShapeDtypeStruct((32, n2), jnp.float32)],
        mesh=_mesh(),
        scratch_types=[pltpu.VMEM((ch,), jnp.int32),
                       pltpu.VMEM((_OPAD,), jnp.float32),
                       pltpu.VMEM((n2,), jnp.float32)],
        compiler_params=_SC_PARAMS,
    )
    def k(oidx_hbm, aidx_hbm, zs_hbm, out_c1, out_c2, idx_v, acc1, acc2):
        c = lax.axis_index("c")
        s = lax.axis_index("s")
        wid = c * _NS + s
        lanes = lax.iota(jnp.int32, 16)
        mask0 = lanes == 0
        ones = jnp.ones((16,), jnp.float32)
        pltpu.sync_copy(zs_hbm.at[pl.ds(0, _OPAD)], acc1)
        pltpu.sync_copy(zs_hbm.at[pl.ds(0, n2)], acc2)

        def count_into(acc, idx_hbm, ep, nb):
            @pl.loop(0, nb)
            def _(i):
                pltpu.sync_copy(idx_hbm.at[pl.ds(wid * ep + i * ch, ch)],
                                idx_v)

                @pl.loop(0, ch // 16)
                def _(e16):
                    vec = idx_v[pl.ds(e16 * 16, 16)]
                    for u in range(16):
                        offs = vec.at[_cidx(u)].get(
                            mode="promise_in_bounds")
                        plsc.addupdate_scatter(acc, [offs], ones, mask=mask0)

        count_into(acc1, oidx_hbm, ep1, nb1)
        count_into(acc2, aidx_hbm, ep2, nb2)
        pltpu.sync_copy(acc1, out_c1.at[wid])
        pltpu.sync_copy(acc2, out_c2.at[wid])

    return k(org_idx, abn_idx, zeros_flat)


def _sc_org_segsum(lab_enh, rel_proj, lab_idx, org_idx, zeros_flat):
    epg = _E1 // 4
    nch = epg // _CH

    @functools.partial(
        pl.kernel,
        out_type=jax.ShapeDtypeStruct((4, 8, _ACCW), jnp.float32),
        mesh=_mesh(),
        scratch_types=[pltpu.VMEM((2, _CH), jnp.int32),
                       pltpu.VMEM((2, _CH), jnp.int32),
                       pltpu.VMEM((4, _SUB), jnp.int32),
                       pltpu.VMEM((4, _SUB), jnp.int32),
                       pltpu.VMEM((4, _SUB, 16), jnp.float32),
                       pltpu.VMEM((4, _SUB, 16), jnp.float32),
                       pltpu.VMEM((_ACCW,), jnp.float32),
                       pltpu.SemaphoreType.DMA,
                       pltpu.SemaphoreType.DMA,
                       pltpu.SemaphoreType.DMA,
                       pltpu.SemaphoreType.DMA,
                       pltpu.SemaphoreType.DMA,
                       pltpu.SemaphoreType.DMA],
        compiler_params=_SC_PARAMS,
    )
    def k(tab_hbm, rel_hbm, lidx_hbm, oidx_hbm, zs_hbm, out_sum,
          lidx_v, oidx_v, gi_v, ri_v, lab_v, rel_v, acc,
          si0, si1, sg0, sg1, sg2, sg3):
        c = lax.axis_index("c")
        s = lax.axis_index("s")
        wid = c * _NS + s
        f = wid % 8
        g = wid // 8
        lanes = lax.iota(jnp.int32, 16)
        lanes8 = lanes * 8
        pltpu.sync_copy(zs_hbm, acc)
        base = g * epg
        si = (si0, si1)
        sg = (sg0, sg1, sg2, sg3)

        def issue_idx(chv, b):
            off = base + chv * _CH
            pltpu.async_copy(lidx_hbm.at[pl.ds(off, _CH)], lidx_v.at[b],
                             si[b])
            pltpu.async_copy(oidx_hbm.at[pl.ds(off, _CH)], oidx_v.at[b],
                             si[b])

        def wait_idx(b):
            pltpu.make_async_copy(lidx_hbm.at[pl.ds(0, _CH)],
                                  lidx_v.at[b], si[b]).wait()
            pltpu.make_async_copy(oidx_hbm.at[pl.ds(0, _CH)],
                                  oidx_v.at[b], si[b]).wait()

        def build_issue(chv, b, j, gb):
            off = base + chv * _CH + j * _SUB

            @pl.loop(0, _SUB // 16)
            def _(c16):
                lvec = lidx_v[b, pl.ds(j * _SUB + c16 * 16, 16)]
                gi_v[gb, pl.ds(c16 * 16, 16)] = lvec * 8 + f
                ri_v[gb, pl.ds(c16 * 16, 16)] = ((off + c16 * 16) * 8 + f
                                                 + lanes8)

            pltpu.async_copy(tab_hbm.at[gi_v.at[gb]], lab_v.at[gb], sg[gb])
            pltpu.async_copy(rel_hbm.at[ri_v.at[gb]], rel_v.at[gb], sg[gb])

        def wait_process(b, j, gb):
            pltpu.make_async_copy(tab_hbm.at[pl.ds(0, _SUB)],
                                  lab_v.at[gb], sg[gb]).wait()
            pltpu.make_async_copy(rel_hbm.at[pl.ds(0, _SUB)],
                                  rel_v.at[gb], sg[gb]).wait()

            @pl.loop(0, _SUB // 16)
            def _(e16):
                vec = oidx_v[b, pl.ds(j * _SUB + e16 * 16, 16)]
                base16 = vec * 16
                es = [e16 * 16 + u for u in range(16)]
                labs = [lab_v[gb, e, :] for e in es]
                rels = [rel_v[gb, e, :] for e in es]
                offs = [base16.at[_cidx(u)].get(
                    mode="promise_in_bounds") + lanes for u in range(16)]
                vals = [a + r for a, r in zip(labs, rels)]
                for u in range(16):
                    plsc.addupdate_scatter(acc, [offs[u]], vals[u])

        issue_idx(0, 0)
        issue_idx(1, 1)

        @pl.loop(0, nch, step=2)
        def _(chv):
            for b in (0, 1):
                cur = chv + b
                wait_idx(b)
                for j in range(3):
                    build_issue(cur, b, j, j)
                for j in range(3, _NSUB):
                    build_issue(cur, b, j, j % 4)
                    wait_process(b, j - 3, (j - 3) % 4)
                for j in range(_NSUB - 3, _NSUB):
                    wait_process(b, j, j % 4)

                @pl.when(cur + 2 < nch)
                def _():
                    issue_idx(cur + 2, b)

        pltpu.sync_copy(acc, out_sum.at[g].at[f])

    return k(lab_enh, rel_proj, lab_idx, org_idx, zeros_flat)


def _sc_abn_segsum(org_tab, org_idx, abn_idx, zeros_flat):
    nch = _E2 // _CH

    @functools.partial(
        pl.kernel,
        out_type=jax.ShapeDtypeStruct((4, 8, _ACCWA), jnp.float32),
        mesh=_mesh(),
        scratch_types=[pltpu.VMEM((2, _CH), jnp.int32),
                       pltpu.VMEM((2, _CH), jnp.int32),
                       pltpu.VMEM((2, _SUB), jnp.int32),
                       pltpu.VMEM((2, _SUB, 16), jnp.float32),
                       pltpu.VMEM((_ACCWA,), jnp.float32),
                       pltpu.VMEM_SHARED((_NO * 8, 16), jnp.float32),
                       pltpu.SemaphoreType.DMA,
                       pltpu.SemaphoreType.DMA,
                       pltpu.SemaphoreType.DMA,
                       pltpu.SemaphoreType.DMA],
        compiler_params=_SC_PARAMS,
    )
    def k(tab_hbm, oidx_hbm, aidx_hbm, zs_hbm, out_sum,
          oidx_v, aidx_v, gi_v, row_v, acc, stab, si0, si1, sg0, sg1):
        c = lax.axis_index("c")
        s = lax.axis_index("s")
        wid = c * _NS + s
        f = wid % 8
        q = wid // 8
        abase = q * _AQ
        lanes = lax.iota(jnp.int32, 16)
        pltpu.sync_copy(zs_hbm.at[pl.ds(0, _ACCWA)], acc)
        trows = _NO * 8 // _NS
        pltpu.sync_copy(tab_hbm.at[pl.ds(s * trows, trows)],
                        stab.at[pl.ds(s * trows, trows)])
        plsc.subcore_barrier()
        si = (si0, si1)
        sg = (sg0, sg1)

        def issue_idx(chv, b):
            off = chv * _CH
            pltpu.async_copy(oidx_hbm.at[pl.ds(off, _CH)], oidx_v.at[b],
                             si[b])
            pltpu.async_copy(aidx_hbm.at[pl.ds(off, _CH)], aidx_v.at[b],
                             si[b])

        def wait_idx(b):
            pltpu.make_async_copy(oidx_hbm.at[pl.ds(0, _CH)],
                                  oidx_v.at[b], si[b]).wait()
            pltpu.make_async_copy(aidx_hbm.at[pl.ds(0, _CH)],
                                  aidx_v.at[b], si[b]).wait()

        def build_issue(chv, b, j, gb):
            @pl.loop(0, _SUB // 16)
            def _(c16):
                ovec = oidx_v[b, pl.ds(j * _SUB + c16 * 16, 16)]
                gi_v[gb, pl.ds(c16 * 16, 16)] = ovec * 8 + f

            pltpu.async_copy(stab.at[gi_v.at[gb]], row_v.at[gb], sg[gb])

        def wait_process(b, j, gb):
            pltpu.make_async_copy(tab_hbm.at[pl.ds(0, _SUB)],
                                  row_v.at[gb], sg[gb]).wait()

            @pl.loop(0, _SUB // 16)
            def _(e16):
                vec = aidx_v[b, pl.ds(j * _SUB + e16 * 16, 16)]
                av = vec - abase
                owned = (av >= 0) & (av < _AQ)
                base16 = jnp.where(owned, av, _AQ) * 16
                es = [e16 * 16 + u for u in range(16)]
                rows = [row_v[gb, e, :] for e in es]
                offs = [base16.at[_cidx(u)].get(
                    mode="promise_in_bounds") + lanes for u in range(16)]
                for u in range(16):
                    plsc.addupdate_scatter(acc, [offs[u]], rows[u])

        issue_idx(0, 0)
        issue_idx(1, 1)

        @pl.loop(0, nch, step=2)
        def _(chv):
            for b in (0, 1):
                cur = chv + b
                wait_idx(b)
                build_issue(cur, b, 0, 0)
                for j in range(1, _NSUB):
                    build_issue(cur, b, j, j % 2)
                    wait_process(b, j - 1, (j - 1) % 2)
                wait_process(b, _NSUB - 1, (_NSUB - 1) % 2)

                @pl.when(cur + 2 < nch)
                def _():
                    issue_idx(cur + 2, b)

        pltpu.sync_copy(acc, out_sum.at[q].at[f])

    return k(org_tab, org_idx, abn_idx, zeros_flat)


def _tc_lab_enh(lab_feats, lab_concept, w1t, w2t, b):
    blk = 1000
    grid = _NL // blk

    def body(x1, x2, w1, w2, bb, o):
        bf = jnp.bfloat16
        acc = jnp.dot(x1[...].astype(bf), w1[...].astype(bf),
                      preferred_element_type=jnp.float32)
        acc = acc + jnp.dot(x2[...].astype(bf), w2[...].astype(bf),
                            preferred_element_type=jnp.float32)
        o[...] = jnp.maximum(acc + bb[...], 0.0)

    return pl.pallas_call(
        body,
        grid=(grid,),
        in_specs=[pl.BlockSpec((blk, _DL), lambda i: (i, 0)),
                  pl.BlockSpec((blk, _DC), lambda i: (i, 0)),
                  pl.BlockSpec((_DL, _DL), lambda i: (0, 0)),
                  pl.BlockSpec((_DC, _DL), lambda i: (0, 0)),
                  pl.BlockSpec((1, _DL), lambda i: (0, 0))],
        out_specs=pl.BlockSpec((blk, _DL), lambda i: (i, 0)),
        out_shape=jax.ShapeDtypeStruct((_NL, _DL), jnp.float32),
    )(lab_feats, lab_concept, w1t, w2t, b)


def _tc_rel_proj(rel_emb, dt):
    blk = 2000
    grid = _E1 // blk

    def body(x, d, o):
        bf = jnp.bfloat16
        o[...] = jnp.dot(x[...].astype(bf), d[...].astype(bf),
                        preferred_element_type=jnp.float32)

    return pl.pallas_call(
        body,
        grid=(grid,),
        in_specs=[pl.BlockSpec((blk, _DC), lambda i: (i, 0)),
                  pl.BlockSpec((_DC, _DL), lambda i: (0, 0))],
        out_specs=pl.BlockSpec((blk, _DL), lambda i: (i, 0)),
        out_shape=jax.ShapeDtypeStruct((_E1, _DL), jnp.float32),
    )(rel_emb, dt)


def _tc_org(parts, cnts, cnts2, wot, wo2t, b1, b2):
    def body(pp, cp, cp2, wo, w2, bb1, bb2, o, o2):
        org_sum = (pp[0, :_NO, :] + pp[1, :_NO, :]
                   + pp[2, :_NO, :] + pp[3, :_NO, :])
        cnt = jnp.sum(cp[...], axis=0)[:_NO, None]
        org_agg = org_sum / jnp.maximum(cnt, 1.0)
        st = jnp.maximum(
            jnp.dot(org_agg, wo[...], preferred_element_type=jnp.float32)
            + bb1[...], 0.0)
        o[...] = jnp.dot(st, w2[...],
                         preferred_element_type=jnp.float32) + bb2[...]
        o2[...] = jnp.sum(cp2[...], axis=0)[:, None]

    return pl.pallas_call(
        body,
        out_shape=[jax.ShapeDtypeStruct((_NO, _DL), jnp.float32),
                   jax.ShapeDtypeStruct((_AQPAD * 4, 1), jnp.float32)],
    )(parts, cnts, cnts2, wot, wo2t, b1, b2)


def _tc_abn(abn_part, cnt_part, abn_feats, abn_concept, a1t, a2t, b):
    blk = 1000
    grid = _NA // blk
    per_q = _AQ // blk

    def body(ap, cp, ff, cc, w1, w2, bb, o):
        msg = ap[0] / jnp.maximum(cp[...], 1.0)
        bf = jnp.bfloat16
        x = (ff[...] + msg).astype(bf)
        o[...] = (jnp.dot(x, w1[...].astype(bf),
                          preferred_element_type=jnp.float32)
                  + jnp.dot(cc[...].astype(bf), w2[...].astype(bf),
                            preferred_element_type=jnp.float32)
                  + bb[...])

    return pl.pallas_call(
        body,
        grid=(grid,),
        in_specs=[
            pl.BlockSpec((1, blk, _DL), lambda i: (i // per_q, i % per_q, 0)),
            pl.BlockSpec((blk, 1), lambda i: (i, 0)),
            pl.BlockSpec((blk, _DL), lambda i: (i, 0)),
            pl.BlockSpec((blk, _DC), lambda i: (i, 0)),
            pl.BlockSpec((_DL, _DL), lambda i: (0, 0)),
            pl.BlockSpec((_DC, _DL), lambda i: (0, 0)),
            pl.BlockSpec((1, _DL), lambda i: (0, 0)),
        ],
        out_specs=pl.BlockSpec((blk, _DL), lambda i: (i, 0)),
        out_shape=jax.ShapeDtypeStruct((_NA, _DL), jnp.float32),
    )(abn_part, cnt_part, abn_feats, abn_concept, a1t, a2t, b)


def kernel(lab_feats, abn_feats, lab_concept, abn_concept, lab_org_rel_emb,
           lab_org_lab_idx, lab_org_org_idx, o2a_abn_idx, o2a_org_idx,
           W_lab_w, W_lab_b, W_abn_w, W_abn_b, W_org_w, W_org_b,
           D_w, W_o2a_w, W_o2a_b):
    f32 = jnp.float32
    w1t = W_lab_w[:, :_DL].T
    w2t = W_lab_w[:, _DL:].T
    a1t = W_abn_w[:, :_DL].T
    a2t = W_abn_w[:, _DL:].T
    dt = D_w.T
    wot = W_org_w.T
    wo2t = W_o2a_w.T
    b_lab = W_lab_b.reshape(1, _DL)
    b_org = W_org_b.reshape(1, _DL)
    b_o2a = W_o2a_b.reshape(1, _DL)
    b_abn = W_abn_b.reshape(1, _DL)

    z_flat = jnp.zeros((_ACCW,), f32)

    c1, c2 = _sc_counts(lab_org_org_idx, o2a_abn_idx, z_flat)
    lab_enh = _tc_lab_enh(lab_feats, lab_concept, w1t, w2t, b_lab)
    rel_proj = _tc_rel_proj(lab_org_rel_emb, dt)
    parts = _sc_org_segsum(lab_enh.reshape(_NL * 8, 16),
                           rel_proj.reshape(_E1 * 8, 16),
                           lab_org_lab_idx, lab_org_org_idx, z_flat)
    parts = (parts.reshape(4, 8, _OPAD, 16).transpose(0, 2, 1, 3)
             .reshape(4, _OPAD, _DL))
    org_tab, cnt2 = _tc_org(parts, c1, c2, wot, wo2t, b_org, b_o2a)
    abn_part = _sc_abn_segsum(org_tab.reshape(_NO * 8, 16),
                              o2a_org_idx, o2a_abn_idx, z_flat)
    abn_part = (abn_part.reshape(4, 8, _AQPAD, 16).transpose(0, 2, 1, 3)
                .reshape(4, _AQPAD, _DL))
    return _tc_abn(abn_part, cnt2, abn_feats, abn_concept, a1t, a2t, b_abn)
